# Initial kernel scaffold; baseline (speedup 1.0000x reference)
import jax
import jax.numpy as jnp
from jax.experimental import pallas as pl


def kernel(hidden_states, w_gate, w_gate_up, w_down, shared_gate_up, shared_down):
    raise NotImplementedError("write your pallas kernel here")


# speedup vs baseline: 1.5803x; 1.5803x over previous
import jax
import jax.numpy as jnp
from jax.experimental import pallas as pl
from jax.experimental.pallas import tpu as pltpu

T = 2048
D = 1024
E = 8
K = 2
I = 512
E9 = E + 1
TM = 1024


def _moe_kernel(x_ref, wg_ref, wgu_ref, wd_ref, out_ref):
    e = pl.program_id(1)
    x = x_ref[...]

    logits = jnp.dot(x, wg_ref[...], preferred_element_type=jnp.float32)
    probs = jax.nn.softmax(logits, axis=-1)
    iota = jax.lax.broadcasted_iota(jnp.int32, (TM, E), 1)
    v1 = jnp.max(probs, axis=1, keepdims=True)
    i1 = jnp.min(jnp.where(probs == v1, iota, E), axis=1, keepdims=True)
    probs2 = jnp.where(iota == i1, -jnp.inf, probs)
    v2 = jnp.max(probs2, axis=1, keepdims=True)
    i2 = jnp.min(jnp.where(probs2 == v2, iota, E), axis=1, keepdims=True)
    coef = jnp.where(i1 == e, v1, 0.0) + jnp.where(i2 == e, v2, 0.0)
    coef = coef / (v1 + v2)
    coef = jnp.where(e == E, jnp.float32(1.0), coef)

    gu = jnp.dot(x, wgu_ref[0], preferred_element_type=jnp.float32)
    g = gu[:, :I]
    u = gu[:, I:]
    h = (g * jax.lax.logistic(g)) * u * coef
    y = jnp.dot(h, wd_ref[0], preferred_element_type=jnp.float32)

    @pl.when(e == 0)
    def _init():
        out_ref[...] = y

    @pl.when(e != 0)
    def _acc():
        out_ref[...] += y


def kernel(hidden_states, w_gate, w_gate_up, w_down, shared_gate_up, shared_down):
    x = hidden_states
    wgu_all = jnp.concatenate([w_gate_up, shared_gate_up[None]], axis=0)
    wd_all = jnp.concatenate([w_down, shared_down[None]], axis=0)

    grid = (T // TM, E9)
    out = pl.pallas_call(
        _moe_kernel,
        grid=grid,
        in_specs=[
            pl.BlockSpec((TM, D), lambda m, e: (m, 0)),
            pl.BlockSpec((D, E), lambda m, e: (0, 0)),
            pl.BlockSpec((1, D, 2 * I), lambda m, e: (e, 0, 0)),
            pl.BlockSpec((1, I, D), lambda m, e: (e, 0, 0)),
        ],
        out_specs=pl.BlockSpec((TM, D), lambda m, e: (m, 0)),
        out_shape=jax.ShapeDtypeStruct((T, D), jnp.float32),
        compiler_params=pltpu.CompilerParams(
            dimension_semantics=("parallel", "arbitrary"),
        ),
    )(x, w_gate, wgu_all, wd_all)
    return out


# speedup vs baseline: 1.6235x; 1.0273x over previous
import jax
import jax.numpy as jnp
from jax.experimental import pallas as pl
from jax.experimental.pallas import tpu as pltpu

T = 2048
D = 1024
E = 8
K = 2
I = 512
E9 = E + 1
TM = 1024


def _moe_kernel(x_ref, wg_ref, wgu_ref, wd_ref, out_ref):
    e = pl.program_id(1)
    x = x_ref[...]

    logits = jnp.dot(x, wg_ref[...], preferred_element_type=jnp.float32)
    probs = jax.nn.softmax(logits, axis=-1)
    iota = jax.lax.broadcasted_iota(jnp.int32, (TM, E), 1)
    v1 = jnp.max(probs, axis=1, keepdims=True)
    i1 = jnp.min(jnp.where(probs == v1, iota, E), axis=1, keepdims=True)
    probs2 = jnp.where(iota == i1, -jnp.inf, probs)
    v2 = jnp.max(probs2, axis=1, keepdims=True)
    i2 = jnp.min(jnp.where(probs2 == v2, iota, E), axis=1, keepdims=True)
    coef = jnp.where(i1 == e, v1, 0.0) + jnp.where(i2 == e, v2, 0.0)
    coef = coef / (v1 + v2)
    coef = jnp.where(e == E, jnp.float32(1.0), coef)

    xb = x.astype(jnp.bfloat16)
    gu = jnp.dot(xb, wgu_ref[0], preferred_element_type=jnp.float32)
    g = gu[:, :I]
    u = gu[:, I:]
    h = (g * jax.lax.logistic(g)) * u * coef
    y = jnp.dot(h.astype(jnp.bfloat16), wd_ref[0],
                preferred_element_type=jnp.float32)

    @pl.when(e == 0)
    def _init():
        out_ref[...] = y

    @pl.when(e != 0)
    def _acc():
        out_ref[...] += y


def kernel(hidden_states, w_gate, w_gate_up, w_down, shared_gate_up, shared_down):
    x = hidden_states
    wgu_all = jnp.concatenate([w_gate_up, shared_gate_up[None]], axis=0)
    wd_all = jnp.concatenate([w_down, shared_down[None]], axis=0)
    wgu_all = wgu_all.astype(jnp.bfloat16)
    wd_all = wd_all.astype(jnp.bfloat16)

    grid = (T // TM, E9)
    out = pl.pallas_call(
        _moe_kernel,
        grid=grid,
        in_specs=[
            pl.BlockSpec((TM, D), lambda m, e: (m, 0)),
            pl.BlockSpec((D, E), lambda m, e: (0, 0)),
            pl.BlockSpec((1, D, 2 * I), lambda m, e: (e, 0, 0)),
            pl.BlockSpec((1, I, D), lambda m, e: (e, 0, 0)),
        ],
        out_specs=pl.BlockSpec((TM, D), lambda m, e: (m, 0)),
        out_shape=jax.ShapeDtypeStruct((T, D), jnp.float32),
        compiler_params=pltpu.CompilerParams(
            dimension_semantics=("parallel", "arbitrary"),
        ),
    )(x, w_gate, wgu_all, wd_all)
    return out


# speedup vs baseline: 1.7882x; 1.1014x over previous
import jax
import jax.numpy as jnp
from jax.experimental import pallas as pl
from jax.experimental.pallas import tpu as pltpu

T = 2048
D = 1024
E = 8
K = 2
I = 512
E9 = E + 1
TM = 1024


def _moe_kernel(x_ref, wg_ref, wgu_ref, wd_ref, out_ref,
                w1_ref, i1_ref, w2_ref, i2_ref):
    e = pl.program_id(1)
    x = x_ref[...]

    @pl.when(e == 0)
    def _route():
        logits = jnp.dot(x, wg_ref[...], preferred_element_type=jnp.float32)
        probs = jax.nn.softmax(logits, axis=-1)
        iota = jax.lax.broadcasted_iota(jnp.int32, (TM, E), 1)
        v1 = jnp.max(probs, axis=1, keepdims=True)
        i1 = jnp.min(jnp.where(probs == v1, iota, E), axis=1, keepdims=True)
        probs2 = jnp.where(iota == i1, -jnp.inf, probs)
        v2 = jnp.max(probs2, axis=1, keepdims=True)
        i2 = jnp.min(jnp.where(probs2 == v2, iota, E), axis=1, keepdims=True)
        denom = v1 + v2
        w1_ref[...] = v1 / denom
        w2_ref[...] = v2 / denom
        i1_ref[...] = i1
        i2_ref[...] = i2

    coef = (jnp.where(i1_ref[...] == e, w1_ref[...], 0.0)
            + jnp.where(i2_ref[...] == e, w2_ref[...], 0.0))
    coef = jnp.where(e == E, jnp.float32(1.0), coef)

    xb = x.astype(jnp.bfloat16)
    gu = jnp.dot(xb, wgu_ref[0], preferred_element_type=jnp.float32)
    g = gu[:, :I]
    u = gu[:, I:]
    h = (g * jax.lax.logistic(g)) * u * coef
    y = jnp.dot(h.astype(jnp.bfloat16), wd_ref[0],
                preferred_element_type=jnp.float32)

    @pl.when(e == 0)
    def _init():
        out_ref[...] = y

    @pl.when(e != 0)
    def _acc():
        out_ref[...] += y


def kernel(hidden_states, w_gate, w_gate_up, w_down, shared_gate_up, shared_down):
    x = hidden_states
    wgu_all = jnp.concatenate([w_gate_up, shared_gate_up[None]], axis=0)
    wd_all = jnp.concatenate([w_down, shared_down[None]], axis=0)
    wgu_all = wgu_all.astype(jnp.bfloat16)
    wd_all = wd_all.astype(jnp.bfloat16)

    grid = (T // TM, E9)
    out = pl.pallas_call(
        _moe_kernel,
        grid=grid,
        in_specs=[
            pl.BlockSpec((TM, D), lambda m, e: (m, 0)),
            pl.BlockSpec((D, E), lambda m, e: (0, 0)),
            pl.BlockSpec((1, D, 2 * I), lambda m, e: (e, 0, 0)),
            pl.BlockSpec((1, I, D), lambda m, e: (e, 0, 0)),
        ],
        out_specs=pl.BlockSpec((TM, D), lambda m, e: (m, 0)),
        out_shape=jax.ShapeDtypeStruct((T, D), jnp.float32),
        scratch_shapes=[
            pltpu.VMEM((TM, 1), jnp.float32),
            pltpu.VMEM((TM, 1), jnp.int32),
            pltpu.VMEM((TM, 1), jnp.float32),
            pltpu.VMEM((TM, 1), jnp.int32),
        ],
        compiler_params=pltpu.CompilerParams(
            dimension_semantics=("parallel", "arbitrary"),
        ),
    )(x, w_gate, wgu_all, wd_all)
    return out
